# baseline (speedup 1.0000x reference)
import functools

import jax
import jax.numpy as jnp
from jax import lax
from jax.experimental import pallas as pl
from jax.experimental.pallas import tpu as pltpu
from jax.experimental.pallas import tpu_sc as plsc

_BATCH = 4
_SEQ = 8192
_D = 1024
_NW = 32
_ROWS_PER_W = _SEQ // _NW
_C = 16
_NCHUNK = _ROWS_PER_W // _C
_LANES = 16
_VECS = _D // _LANES


def _sc_body(x_hbm, pos_hbm, out_hbm, pos_v, x_v):
    wid = lax.axis_index("s") * 2 + lax.axis_index("c")
    base = wid * _ROWS_PER_W

    def chunk_body(ci, carry):
        row0 = base + ci * _C
        pltpu.sync_copy(pos_hbm.at[pl.ds(row0, _C)], pos_v)
        for b in range(_BATCH):
            pltpu.sync_copy(x_hbm.at[b, pl.ds(row0, _C)], x_v)

            def row_body(i, c2):
                def vec_body(j, c3):
                    sl = pl.ds(j * _LANES, _LANES)
                    x_v[i, sl] = x_v[i, sl] + pos_v[i, sl]
                    return c3

                return lax.fori_loop(0, _VECS, vec_body, c2, unroll=8)

            lax.fori_loop(0, _C, row_body, 0)
            pltpu.sync_copy(x_v, out_hbm.at[b, pl.ds(row0, _C)])
        return carry

    lax.fori_loop(0, _NCHUNK, chunk_body, 0)


def kernel(x, pos_table):
    mesh = plsc.VectorSubcoreMesh(core_axis_name="c", subcore_axis_name="s")
    fn = functools.partial(
        pl.kernel,
        mesh=mesh,
        out_type=jax.ShapeDtypeStruct((_BATCH, _SEQ, _D), jnp.float32),
        scratch_types=[
            pltpu.VMEM((_C, _D), jnp.float32),
            pltpu.VMEM((_C, _D), jnp.float32),
        ],
    )(_sc_body)
    return fn(x, pos_table)


# speedup vs baseline: 4.4683x; 4.4683x over previous
import jax
import jax.numpy as jnp
from jax.experimental import pallas as pl


_S_BLK = 512


def _add_kernel(x_ref, pos_ref, o_ref):
    o_ref[...] = x_ref[...] + pos_ref[...]


def kernel(x, pos_table):
    batch, seq_len, d_model = x.shape
    grid = (seq_len // _S_BLK, batch)
    return pl.pallas_call(
        _add_kernel,
        grid=grid,
        in_specs=[
            pl.BlockSpec((1, _S_BLK, d_model), lambda s, b: (b, s, 0)),
            pl.BlockSpec((_S_BLK, d_model), lambda s, b: (s, 0)),
        ],
        out_specs=pl.BlockSpec((1, _S_BLK, d_model), lambda s, b: (b, s, 0)),
        out_shape=jax.ShapeDtypeStruct(x.shape, x.dtype),
    )(x, pos_table)


# speedup vs baseline: 5.1776x; 1.1587x over previous
import jax
import jax.numpy as jnp
from jax.experimental import pallas as pl


_S_BLK = 512


def _add_kernel(x_ref, pos_ref, o_ref):
    o_ref[...] = x_ref[...] + pos_ref[...][None]


def kernel(x, pos_table):
    batch, seq_len, d_model = x.shape
    grid = (seq_len // _S_BLK,)
    return pl.pallas_call(
        _add_kernel,
        grid=grid,
        in_specs=[
            pl.BlockSpec((batch, _S_BLK, d_model), lambda s: (0, s, 0)),
            pl.BlockSpec((_S_BLK, d_model), lambda s: (s, 0)),
        ],
        out_specs=pl.BlockSpec((batch, _S_BLK, d_model), lambda s: (0, s, 0)),
        out_shape=jax.ShapeDtypeStruct(x.shape, x.dtype),
    )(x, pos_table)


# speedup vs baseline: 5.1860x; 1.0016x over previous
import jax
import jax.numpy as jnp
from jax.experimental import pallas as pl


_S_BLK = 256


def _add_kernel(x_ref, pos_ref, o_ref):
    o_ref[...] = x_ref[...] + pos_ref[...][None]


def kernel(x, pos_table):
    batch, seq_len, d_model = x.shape
    grid = (seq_len // _S_BLK,)
    return pl.pallas_call(
        _add_kernel,
        grid=grid,
        in_specs=[
            pl.BlockSpec((batch, _S_BLK, d_model), lambda s: (0, s, 0)),
            pl.BlockSpec((_S_BLK, d_model), lambda s: (s, 0)),
        ],
        out_specs=pl.BlockSpec((batch, _S_BLK, d_model), lambda s: (0, s, 0)),
        out_shape=jax.ShapeDtypeStruct(x.shape, x.dtype),
    )(x, pos_table)


# speedup vs baseline: 5.2109x; 1.0048x over previous
import jax
import jax.numpy as jnp
from jax.experimental import pallas as pl
from jax.experimental.pallas import tpu as pltpu


_S_BLK = 512


def _add_kernel(x_ref, pos_ref, o_ref):
    o_ref[...] = x_ref[...] + pos_ref[...][None]


def kernel(x, pos_table):
    batch, seq_len, d_model = x.shape
    grid = (seq_len // _S_BLK,)
    return pl.pallas_call(
        _add_kernel,
        grid=grid,
        in_specs=[
            pl.BlockSpec((batch, _S_BLK, d_model), lambda s: (0, s, 0)),
            pl.BlockSpec((_S_BLK, d_model), lambda s: (s, 0)),
        ],
        out_specs=pl.BlockSpec((batch, _S_BLK, d_model), lambda s: (0, s, 0)),
        out_shape=jax.ShapeDtypeStruct(x.shape, x.dtype),
        compiler_params=pltpu.CompilerParams(
            dimension_semantics=("parallel",),
        ),
    )(x, pos_table)
